# baseline (speedup 1.0000x reference)
import functools

import jax
import jax.numpy as jnp
from jax import lax
from jax.experimental import pallas as pl
from jax.experimental.pallas import tpu as pltpu
from jax.experimental.pallas import tpu_sc as plsc

_NC, _NS, _L = 2, 16, 16
_NW = _NC * _NS
_D = 256
_B = 64 * 1024
_ROWS = 2 * _B
_RPW = _ROWS // _NW
_CH = 128
_NCH = _RPW // _CH
_IDXROWS = _RPW // _CH


@functools.partial(
    pl.kernel,
    mesh=plsc.VectorSubcoreMesh(core_axis_name="c", subcore_axis_name="s"),
    out_type=jax.ShapeDtypeStruct((_ROWS, _D), jnp.float32),
    scratch_types=[
        pltpu.VMEM((_IDXROWS, _CH), jnp.int32),
        pltpu.VMEM((_CH, _D), jnp.float32),
        pltpu.VMEM((_CH, _D), jnp.float32),
        pltpu.SemaphoreType.DMA,
        pltpu.SemaphoreType.DMA,
        pltpu.SemaphoreType.DMA,
        pltpu.SemaphoreType.DMA,
    ],
)
def _sc_lookup(idx_hbm, table_hbm, out_hbm, idx_v, buf0, buf1,
               sg0, sg1, so0, so1):
    bufs = (buf0, buf1)
    sgs = (sg0, sg1)
    sos = (so0, so1)
    wid = lax.axis_index("s") * _NC + lax.axis_index("c")
    base = wid * _RPW

    pltpu.sync_copy(idx_hbm.at[pl.ds(wid * _IDXROWS, _IDXROWS)], idx_v)

    off = (lax.iota(jnp.int32, _L) % 2) * 64

    def add_row(i, carry):
        def add_vec(j, c2):
            sl = pl.ds(j * _L, _L)
            idx_v[i, sl] = idx_v[i, sl] + off
            return c2
        return lax.fori_loop(0, _CH // _L, add_vec, carry)

    lax.fori_loop(0, _IDXROWS, add_row, 0)

    def start_gather(c, b):
        pltpu.async_copy(table_hbm.at[idx_v.at[c]], bufs[b], sgs[b])

    def wait_gather(b):
        pltpu.make_async_copy(out_hbm.at[pl.ds(0, _CH)], bufs[b], sgs[b]).wait()

    def out_desc(c, b):
        return pltpu.make_async_copy(
            bufs[b], out_hbm.at[pl.ds(base + c * _CH, _CH)], sos[b])

    start_gather(0, 0)
    start_gather(1, 1)

    def pair(s, carry):
        for b in range(2):
            c = 2 * s + b
            wait_gather(b)
            out_desc(c, b).start()
            out_desc(c, b).wait()
            start_gather(c + 2, b)
        return carry

    lax.fori_loop(0, _NCH // 2 - 1, pair, 0)

    for b in range(2):
        c = _NCH - 2 + b
        wait_gather(b)
        out_desc(c, b).start()
        out_desc(c, b).wait()


def kernel(position_inds, col_embed, row_embed):
    table = jnp.concatenate([col_embed, row_embed], axis=0)
    idx = position_inds.astype(jnp.int32).reshape(_ROWS // _CH, _CH)
    out = _sc_lookup(idx, table)
    return out.reshape(64, 1024, 2 * _D)


# speedup vs baseline: 1.7742x; 1.7742x over previous
import functools

import jax
import jax.numpy as jnp
from jax import lax
from jax.experimental import pallas as pl
from jax.experimental.pallas import tpu as pltpu
from jax.experimental.pallas import tpu_sc as plsc

_NC, _NS = 2, 16
_NW = _NC * _NS
_D = 256
_P = 64 * 1024
_PPW = _P // _NW
_CH = 64
_NCH = _PPW // _CH
_IDXROWS = _PPW // _CH


@functools.partial(
    pl.kernel,
    mesh=plsc.VectorSubcoreMesh(core_axis_name="c", subcore_axis_name="s"),
    out_type=jax.ShapeDtypeStruct((_P, 2 * _D), jnp.float32),
    scratch_types=[
        pltpu.VMEM((_IDXROWS, _CH), jnp.int32),
        pltpu.VMEM((_IDXROWS, _CH), jnp.int32),
        pltpu.VMEM((_CH, 2 * _D), jnp.float32),
        pltpu.VMEM((_CH, 2 * _D), jnp.float32),
        pltpu.SemaphoreType.DMA,
        pltpu.SemaphoreType.DMA,
        pltpu.SemaphoreType.DMA,
        pltpu.SemaphoreType.DMA,
    ],
)
def _sc_lookup(idx_x_hbm, idx_y_hbm, col_hbm, row_hbm, out_hbm,
               idxx_v, idxy_v, buf0, buf1, sg0, sg1, so0, so1):
    bufs = (buf0, buf1)
    sgs = (sg0, sg1)
    sos = (so0, so1)
    wid = lax.axis_index("s") * _NC + lax.axis_index("c")
    base = wid * _PPW

    pltpu.sync_copy(idx_x_hbm.at[pl.ds(wid * _IDXROWS, _IDXROWS)], idxx_v)
    pltpu.sync_copy(idx_y_hbm.at[pl.ds(wid * _IDXROWS, _IDXROWS)], idxy_v)

    def start_gathers(c, b):
        pltpu.async_copy(
            col_hbm.at[idxx_v.at[c]], bufs[b].at[:, pl.ds(0, _D)], sgs[b])
        pltpu.async_copy(
            row_hbm.at[idxy_v.at[c]], bufs[b].at[:, pl.ds(_D, _D)], sgs[b])

    def wait_gathers(b):
        pltpu.make_async_copy(
            col_hbm.at[idxx_v.at[0]], bufs[b].at[:, pl.ds(0, _D)], sgs[b]).wait()
        pltpu.make_async_copy(
            row_hbm.at[idxy_v.at[0]], bufs[b].at[:, pl.ds(_D, _D)], sgs[b]).wait()

    def out_desc(c, b):
        return pltpu.make_async_copy(
            bufs[b], out_hbm.at[pl.ds(base + c * _CH, _CH)], sos[b])

    start_gathers(0, 0)
    start_gathers(1, 1)

    def pair(s, carry):
        for b in range(2):
            c = 2 * s + b
            wait_gathers(b)
            out_desc(c, b).start()
            out_desc(c, b).wait()
            start_gathers(c + 2, b)
        return carry

    lax.fori_loop(0, _NCH // 2 - 1, pair, 0)

    for b in range(2):
        c = _NCH - 2 + b
        wait_gathers(b)
        out_desc(c, b).start()
        out_desc(c, b).wait()


def kernel(position_inds, col_embed, row_embed):
    pi = position_inds.astype(jnp.int32)
    idx_x = pi[:, :, 0].reshape(_P // _CH, _CH)
    idx_y = pi[:, :, 1].reshape(_P // _CH, _CH)
    out = _sc_lookup(idx_x, idx_y, col_embed, row_embed)
    return out.reshape(64, 1024, 2 * _D)
